# baseline (speedup 1.0000x reference)
import functools

import numpy as np
import jax
import jax.numpy as jnp
from jax import lax
from jax.experimental import pallas as pl
from jax.experimental.pallas import tpu as pltpu
from jax.experimental.pallas import tpu_sc as plsc

_N = 65536
_L = 16
_F = 2
_T = 1 << 19
_BASE_RES = 16.0
_MAX_RES = 512.0
_GROWTH = float(np.exp((np.log(_MAX_RES) - np.log(_BASE_RES)) / (_L - 1)))
_SCALES = np.array([np.float32(_BASE_RES * (_GROWTH ** l)) for l in range(_L)],
                   dtype=np.float32)
_PRIMES = [1, 2654435761 - (1 << 32), 805459861, 3674653429 - (1 << 32)]

_NW = 32
_PW = _N // _NW
_C = 1024
_NCHUNK = _PW // _C
_NGRP = _C // 16


def _encode_body(coords_hbm, tables_hbm, scales_hbm, out_hbm,
                 coords_v, scales_v, idx0_v, idx1_v, rows0_v, rows1_v,
                 feats_v, sem0, sem1):
    cid = lax.axis_index("c")
    sid = lax.axis_index("s")
    wid = sid * 2 + cid
    base = wid * _PW
    for d in range(4):
        pltpu.sync_copy(coords_hbm.at[pl.ds(d * _N + base, _PW)],
                        coords_v.at[pl.ds(d * _PW, _PW)])
    pltpu.sync_copy(scales_hbm, scales_v)
    io = lax.iota(jnp.int32, 16)
    io32 = io * 32
    col0 = jnp.zeros((16,), jnp.int32)

    def chunk_body(ch, carry):
        cbase = ch * _C

        def level_body(l, carry):
            res = plsc.load_gather(scales_v, [col0 + l])
            lvl_off2 = l * (2 * _T)

            def grp_a(g, carry):
                p0 = cbase + g * 16
                hs = []
                for d in range(4):
                    cv = coords_v[pl.ds(d * _PW + p0, 16)]
                    scaled = cv * res
                    ip = scaled.astype(jnp.int32)
                    if d == 0:
                        a = ip
                        b = ip + 1
                    else:
                        a = ip * _PRIMES[d]
                        b = a + _PRIMES[d]
                    hs.append((a, b))
                s0 = g * 16
                for c in range(16):
                    h = hs[0][c & 1]
                    for d in range(1, 4):
                        h = h ^ hs[d][(c >> d) & 1]
                    i0 = (h & (_T - 1)) * 2 + lvl_off2
                    idx0_v[pl.ds(c * _C + s0, 16)] = i0
                    idx1_v[pl.ds(c * _C + s0, 16)] = i0 + 1
                return carry

            lax.fori_loop(0, _NGRP, grp_a, 0)
            cp0 = pltpu.async_copy(tables_hbm.at[idx0_v], rows0_v, sem0)
            cp1 = pltpu.async_copy(tables_hbm.at[idx1_v], rows1_v, sem1)
            cp0.wait()
            cp1.wait()

            def grp_c(g, carry):
                p0 = cbase + g * 16
                w = []
                omw = []
                for d in range(4):
                    cv = coords_v[pl.ds(d * _PW + p0, 16)]
                    scaled = cv * res
                    ip = scaled.astype(jnp.int32)
                    wd = scaled - ip.astype(jnp.float32)
                    w.append(wd)
                    omw.append(1.0 - wd)
                p_a = [omw[0] * omw[1], w[0] * omw[1], omw[0] * w[1], w[0] * w[1]]
                p_b = [omw[2] * omw[3], w[2] * omw[3], omw[2] * w[3], w[2] * w[3]]
                acc0 = jnp.zeros((16,), jnp.float32)
                acc1 = jnp.zeros((16,), jnp.float32)
                s0 = g * 16
                for c in range(16):
                    wc = p_a[c & 3] * p_b[(c >> 2) & 3]
                    f0 = rows0_v[pl.ds(c * _C + s0, 16)]
                    f1 = rows1_v[pl.ds(c * _C + s0, 16)]
                    acc0 = acc0 + f0 * wc
                    acc1 = acc1 + f1 * wc
                fs = io32 + (g * (16 * 2 * _L) + 2 * l)
                plsc.store_scatter(feats_v, [fs], acc0)
                plsc.store_scatter(feats_v, [fs + 1], acc1)
                return carry

            lax.fori_loop(0, _NGRP, grp_c, 0)
            return carry

        lax.fori_loop(0, _L, level_body, 0)
        pltpu.sync_copy(feats_v,
                        out_hbm.at[pl.ds((base + cbase) * (2 * _L), _C * 2 * _L)])
        return carry

    lax.fori_loop(0, _NCHUNK, chunk_body, 0)


_encode = functools.partial(
    pl.kernel,
    mesh=plsc.VectorSubcoreMesh(core_axis_name="c", subcore_axis_name="s"),
    out_type=jax.ShapeDtypeStruct((_N * 2 * _L,), jnp.float32),
    compiler_params=pltpu.CompilerParams(
        needs_layout_passes=False, use_tc_tiling_on_sc=False),
    scratch_types=[
        pltpu.VMEM((4 * _PW,), jnp.float32),
        pltpu.VMEM((_L,), jnp.float32),
        pltpu.VMEM((16 * _C,), jnp.int32),
        pltpu.VMEM((16 * _C,), jnp.int32),
        pltpu.VMEM((16 * _C,), jnp.float32),
        pltpu.VMEM((16 * _C,), jnp.float32),
        pltpu.VMEM((_C * 2 * _L,), jnp.float32),
        pltpu.SemaphoreType.DMA,
        pltpu.SemaphoreType.DMA,
    ],
)(_encode_body)


_BN = 4096


def _mlp_body(x_ref, w1_ref, b1_ref, w2_ref, b2_ref, w3_ref, b3_ref, o_ref):
    x = x_ref[...]
    h = jnp.maximum(
        jnp.dot(x, w1_ref[...], preferred_element_type=jnp.float32) + b1_ref[...],
        0.0)
    h = jnp.maximum(
        jnp.dot(h, w2_ref[...], preferred_element_type=jnp.float32) + b2_ref[...],
        0.0)
    z = jnp.dot(h, w3_ref[...], preferred_element_type=jnp.float32) + b3_ref[...]
    o_ref[...] = jax.nn.sigmoid(z)


def _mlp(feats, W1, b1, W2, b2, W3p, b3p):
    return pl.pallas_call(
        _mlp_body,
        grid=(_N // _BN,),
        in_specs=[
            pl.BlockSpec((_BN, 2 * _L), lambda i: (i, 0)),
            pl.BlockSpec((2 * _L, 256), lambda i: (0, 0)),
            pl.BlockSpec((1, 256), lambda i: (0, 0)),
            pl.BlockSpec((256, 128), lambda i: (0, 0)),
            pl.BlockSpec((1, 128), lambda i: (0, 0)),
            pl.BlockSpec((128, 128), lambda i: (0, 0)),
            pl.BlockSpec((1, 128), lambda i: (0, 0)),
        ],
        out_specs=pl.BlockSpec((_BN, 128), lambda i: (i, 0)),
        out_shape=jax.ShapeDtypeStruct((_N, 128), jnp.float32),
    )(feats, W1, b1, W2, b2, W3p, b3p)


def kernel(coords, tables, W1, b1, W2, b2, W3, b3):
    coords_flat = coords.T.reshape(-1)
    tables_flat = tables.reshape(-1)
    scales = jnp.asarray(_SCALES)
    feats = _encode(coords_flat, tables_flat, scales).reshape(_N, 2 * _L)
    W3p = jnp.pad(W3, ((0, 0), (0, 125)))
    b3p = jnp.pad(b3, (0, 125))
    out = _mlp(feats, W1, b1[None, :], W2, b2[None, :], W3p, b3p[None, :])
    return out[:, :3]
